# baseline (speedup 1.0000x reference)
import functools

import numpy as np
import jax
import jax.numpy as jnp
from jax import lax
from jax.experimental import pallas as pl
from jax.experimental.pallas import tpu as pltpu
from jax.experimental.pallas import tpu_sc as plsc

_SR_MAX_TAU = 40
_LAGS = 2 * _SR_MAX_TAU + 1
_T = 4096
_KF = _T // 2 + 1
_KPAD = 17 * 128
_LPAD = 128
_NW = 32
_L = 16
_QB = 8


def _idft_consts():
    k = np.arange(_KF)
    t = (np.arange(_LAGS) - _SR_MAX_TAU) % _T
    theta = 2.0 * np.pi * ((k[:, None] * t[None, :]) % _T) / _T
    w = np.full((_KF, 1), 2.0)
    w[0, 0] = 1.0
    w[-1, 0] = 1.0
    c = (w * np.cos(theta)) / _T
    s = (-w * np.sin(theta)) / _T
    cp = np.zeros((_KPAD, _LPAD), np.float32)
    sp = np.zeros((_KPAD, _LPAD), np.float32)
    cp[:_KF, :_LAGS] = c
    sp[:_KF, :_LAGS] = s
    return cp, sp


def _pairs(x, m):
    a = jnp.concatenate(
        [jnp.broadcast_to(x[k:k + 1], (m - 1 - k, x.shape[1]))
         for k in range(m - 1)], axis=0)
    b = jnp.concatenate([x[k + 1:m] for k in range(m - 1)], axis=0)
    return a, b


def _whiten_idft_body(xr_ref, xi_ref, c_ref, s_ref, out_ref):
    m = xr_ref.shape[1]
    prs, pis = [], []
    for q in range(_QB):
        xr = xr_ref[q]
        xi = xi_ref[q]
        ar, br = _pairs(xr, m)
        ai, bi = _pairs(xi, m)
        gr = ar * br + ai * bi
        gi = ai * br - ar * bi
        inv = 1.0 / (jnp.sqrt(gr * gr + gi * gi) + 1e-12)
        prs.append(gr * inv)
        pis.append(gi * inv)
    pr = jnp.concatenate(prs, axis=0)
    pi = jnp.concatenate(pis, axis=0)
    out_ref[:, :] = (
        jnp.dot(pr, c_ref[:, :], precision=lax.Precision.HIGHEST,
                preferred_element_type=jnp.float32)
        + jnp.dot(pi, s_ref[:, :], precision=lax.Precision.HIGHEST,
                  preferred_element_type=jnp.float32)
    )


def _whiten_idft(xr, xi, cmat, smat):
    b_sz, m_sz, _ = xr.shape
    p_sz = m_sz * (m_sz - 1) // 2
    rows_blk = _QB * p_sz
    spec_x = pl.BlockSpec((_QB, m_sz, _KPAD), lambda i: (i, 0, 0))
    spec_c = pl.BlockSpec((_KPAD, _LPAD), lambda i: (0, 0))
    return pl.pallas_call(
        _whiten_idft_body,
        grid=(b_sz // _QB,),
        in_specs=[spec_x, spec_x, spec_c, spec_c],
        out_specs=pl.BlockSpec((rows_blk, _LPAD), lambda i: (i, 0)),
        out_shape=jax.ShapeDtypeStruct((b_sz * p_sz, _LPAD), jnp.float32),
    )(xr, xi, cmat, smat)


def _make_sc_search(n_grid, n_pairs, batch, table_cols):
    npw_raw = -(-n_grid // _NW)
    npw = -(-npw_raw // _L) * _L
    chunks = npw // _L
    n_pad = npw * _NW
    mesh = plsc.VectorSubcoreMesh(core_axis_name="c", subcore_axis_name="s")

    @functools.partial(
        pl.kernel,
        mesh=mesh,
        compiler_params=pltpu.CompilerParams(needs_layout_passes=False),
        out_type=(
            jax.ShapeDtypeStruct((_NW, batch, _L), jnp.float32),
            jax.ShapeDtypeStruct((_NW, batch, _L), jnp.int32),
        ),
        scratch_types=[
            pltpu.VMEM((batch * table_cols,), jnp.float32),
            pltpu.VMEM((npw * n_pairs,), jnp.int32),
            pltpu.VMEM((batch, _L), jnp.float32),
            pltpu.VMEM((batch, _L), jnp.int32),
        ],
    )
    def sc_search(table_hbm, tau_hbm, omax_hbm, oidx_hbm,
                  table_v, tau_v, rmax_v, ridx_v):
        wid = lax.axis_index("s") * 2 + lax.axis_index("c")
        pltpu.sync_copy(table_hbm, table_v)
        pltpu.sync_copy(tau_hbm.at[pl.ds(wid * (npw * n_pairs), npw * n_pairs)],
                        tau_v)
        for b in range(batch):
            rmax_v[b, :] = jnp.full((_L,), -jnp.inf, jnp.float32)
            ridx_v[b, :] = jnp.zeros((_L,), jnp.int32)
        base = wid * npw
        lane28 = lax.iota(jnp.int32, _L) * n_pairs

        def chunk_body(c, _):
            posbase = lane28 + c * (_L * n_pairs)
            accs = [jnp.zeros((_L,), jnp.float32) for _ in range(batch)]
            for p in range(n_pairs):
                tval = plsc.load_gather(tau_v, [posbase + p])
                idx = tval + p * _LAGS
                for b in range(batch):
                    accs[b] = accs[b] + plsc.load_gather(
                        table_v, [idx + b * table_cols])
            nvec = base + c * _L + lax.iota(jnp.int32, _L)
            valid = nvec < n_grid
            for b in range(batch):
                m = rmax_v[b, :]
                upd = jnp.logical_and(accs[b] > m, valid)
                rmax_v[b, :] = jnp.where(upd, accs[b], m)
                ridx_v[b, :] = jnp.where(upd, nvec, ridx_v[b, :])
            return _

        lax.fori_loop(0, chunks, chunk_body, None)
        pltpu.sync_copy(rmax_v, omax_hbm.at[wid])
        pltpu.sync_copy(ridx_v, oidx_hbm.at[wid])

    return sc_search, npw, n_pad


def kernel(signal, x_grid, rec_centroid, tau, combinations):
    del combinations
    b_sz, m_sz, t_sz = signal.shape
    p_sz = m_sz * (m_sz - 1) // 2
    n_grid = tau.shape[0]
    table_cols = p_sz * _LAGS

    spec = jnp.fft.rfft(signal, axis=-1)
    xr = jnp.pad(jnp.real(spec).astype(jnp.float32),
                 ((0, 0), (0, 0), (0, _KPAD - _KF)))
    xi = jnp.pad(jnp.imag(spec).astype(jnp.float32),
                 ((0, 0), (0, 0), (0, _KPAD - _KF)))

    cnp, snp = _idft_consts()
    cc = _whiten_idft(xr, xi, jnp.asarray(cnp), jnp.asarray(snp))
    table = cc[:, :_LAGS].reshape(b_sz * table_cols)

    sc_search, npw, n_pad = _make_sc_search(n_grid, p_sz, b_sz, table_cols)
    tau_pad = jnp.pad(tau.astype(jnp.int32),
                      ((0, n_pad - n_grid), (0, 0))).reshape(-1)
    pmax, pidx = sc_search(table, tau_pad)

    vals = pmax.transpose(1, 0, 2).reshape(b_sz, _NW * _L)
    idxs = pidx.transpose(1, 0, 2).reshape(b_sz, _NW * _L)
    mx = vals.max(axis=1, keepdims=True)
    best = jnp.where(vals == mx, idxs, jnp.int32(2**31 - 1)).min(axis=1)
    return x_grid[best] - rec_centroid[None, :]


# speedup vs baseline: 1.4822x; 1.4822x over previous
import functools

import numpy as np
import jax
import jax.numpy as jnp
from jax import lax
from jax.experimental import pallas as pl
from jax.experimental.pallas import tpu as pltpu
from jax.experimental.pallas import tpu_sc as plsc

_SR_MAX_TAU = 40
_LAGS = 2 * _SR_MAX_TAU + 1
_T = 4096
_KF = _T // 2 + 1
_KPAD = 17 * 128
_LPAD = 128
_NW = 32
_L = 16
_QB = 8


def _idft_consts():
    k = np.arange(_KF)
    t = (np.arange(_LAGS) - _SR_MAX_TAU) % _T
    theta = 2.0 * np.pi * ((k[:, None] * t[None, :]) % _T) / _T
    w = np.full((_KF, 1), 2.0)
    w[0, 0] = 1.0
    w[-1, 0] = 1.0
    c = (w * np.cos(theta)) / _T
    s = (-w * np.sin(theta)) / _T
    cp = np.zeros((_KPAD, _LPAD), np.float32)
    sp = np.zeros((_KPAD, _LPAD), np.float32)
    cp[:_KF, :_LAGS] = c
    sp[:_KF, :_LAGS] = s
    return cp, sp


def _pairs(x, m):
    a = jnp.concatenate(
        [jnp.broadcast_to(x[k:k + 1], (m - 1 - k, x.shape[1]))
         for k in range(m - 1)], axis=0)
    b = jnp.concatenate([x[k + 1:m] for k in range(m - 1)], axis=0)
    return a, b


def _whiten_idft_body(xr_ref, xi_ref, c_ref, s_ref, out_ref):
    m = xr_ref.shape[1]
    prs, pis = [], []
    for q in range(_QB):
        xr = xr_ref[q]
        xi = xi_ref[q]
        ar, br = _pairs(xr, m)
        ai, bi = _pairs(xi, m)
        gr = ar * br + ai * bi
        gi = ai * br - ar * bi
        inv = 1.0 / (jnp.sqrt(gr * gr + gi * gi) + 1e-12)
        prs.append(gr * inv)
        pis.append(gi * inv)
    pr = jnp.concatenate(prs, axis=0)
    pi = jnp.concatenate(pis, axis=0)
    out_ref[:, :] = (
        jnp.dot(pr, c_ref[:, :], precision=lax.Precision.HIGHEST,
                preferred_element_type=jnp.float32)
        + jnp.dot(pi, s_ref[:, :], precision=lax.Precision.HIGHEST,
                  preferred_element_type=jnp.float32)
    )


def _whiten_idft(xr, xi, cmat, smat):
    b_sz, m_sz, _ = xr.shape
    p_sz = m_sz * (m_sz - 1) // 2
    rows_blk = _QB * p_sz
    spec_x = pl.BlockSpec((_QB, m_sz, _KPAD), lambda i: (i, 0, 0))
    spec_c = pl.BlockSpec((_KPAD, _LPAD), lambda i: (0, 0))
    return pl.pallas_call(
        _whiten_idft_body,
        grid=(b_sz // _QB,),
        in_specs=[spec_x, spec_x, spec_c, spec_c],
        out_specs=pl.BlockSpec((rows_blk, _LPAD), lambda i: (i, 0)),
        out_shape=jax.ShapeDtypeStruct((b_sz * p_sz, _LPAD), jnp.float32),
    )(xr, xi, cmat, smat)


def _make_sc_search(n_grid, n_pairs, batch, table_cols):
    npw_raw = -(-n_grid // _NW)
    npw = -(-npw_raw // _L) * _L
    chunks = npw // _L
    n_pad = npw * _NW
    mesh = plsc.VectorSubcoreMesh(core_axis_name="c", subcore_axis_name="s")

    @functools.partial(
        pl.kernel,
        mesh=mesh,
        compiler_params=pltpu.CompilerParams(needs_layout_passes=False),
        out_type=(
            jax.ShapeDtypeStruct((_NW, batch, _L), jnp.float32),
            jax.ShapeDtypeStruct((_NW, batch, _L), jnp.int32),
        ),
        scratch_types=[
            pltpu.VMEM((batch * table_cols,), jnp.float32),
            pltpu.VMEM((npw * n_pairs,), jnp.int32),
            pltpu.VMEM((batch, _L), jnp.float32),
            pltpu.VMEM((batch, _L), jnp.int32),
        ],
    )
    def sc_search(table_hbm, tau_hbm, omax_hbm, oidx_hbm,
                  table_v, tau_v, rmax_v, ridx_v):
        wid = lax.axis_index("s") * 2 + lax.axis_index("c")
        pltpu.sync_copy(table_hbm, table_v)
        pltpu.sync_copy(tau_hbm.at[pl.ds(wid * (npw * n_pairs), npw * n_pairs)],
                        tau_v)
        for b in range(batch):
            rmax_v[b, :] = jnp.full((_L,), -jnp.inf, jnp.float32)
            ridx_v[b, :] = jnp.zeros((_L,), jnp.int32)
        base = wid * npw
        lane28 = lax.iota(jnp.int32, _L) * n_pairs

        def chunk_body(c, _):
            posbase = lane28 + c * (_L * n_pairs)

            def pair_body(p, accs):
                tval = plsc.load_gather(tau_v, [posbase + p])
                idx = tval + p * _LAGS
                return tuple(
                    accs[b] + plsc.load_gather(table_v, [idx + b * table_cols])
                    for b in range(batch)
                )

            zeros = tuple(jnp.zeros((_L,), jnp.float32) for _ in range(batch))
            accs = lax.fori_loop(0, n_pairs, pair_body, zeros)
            nvec = base + c * _L + lax.iota(jnp.int32, _L)
            valid = nvec < n_grid
            for b in range(batch):
                m = rmax_v[b, :]
                upd = jnp.logical_and(accs[b] > m, valid)
                rmax_v[b, :] = jnp.where(upd, accs[b], m)
                ridx_v[b, :] = jnp.where(upd, nvec, ridx_v[b, :])
            return _

        lax.fori_loop(0, chunks, chunk_body, None)
        pltpu.sync_copy(rmax_v, omax_hbm.at[wid])
        pltpu.sync_copy(ridx_v, oidx_hbm.at[wid])

    return sc_search, npw, n_pad


def kernel(signal, x_grid, rec_centroid, tau, combinations):
    del combinations
    b_sz, m_sz, t_sz = signal.shape
    p_sz = m_sz * (m_sz - 1) // 2
    n_grid = tau.shape[0]
    table_cols = p_sz * _LAGS

    spec = jnp.fft.rfft(signal, axis=-1)
    xr = jnp.pad(jnp.real(spec).astype(jnp.float32),
                 ((0, 0), (0, 0), (0, _KPAD - _KF)))
    xi = jnp.pad(jnp.imag(spec).astype(jnp.float32),
                 ((0, 0), (0, 0), (0, _KPAD - _KF)))

    cnp, snp = _idft_consts()
    cc = _whiten_idft(xr, xi, jnp.asarray(cnp), jnp.asarray(snp))
    table = cc[:, :_LAGS].reshape(b_sz * table_cols)

    sc_search, npw, n_pad = _make_sc_search(n_grid, p_sz, b_sz, table_cols)
    tau_pad = jnp.pad(tau.astype(jnp.int32),
                      ((0, n_pad - n_grid), (0, 0))).reshape(-1)
    pmax, pidx = sc_search(table, tau_pad)

    vals = pmax.transpose(1, 0, 2).reshape(b_sz, _NW * _L)
    idxs = pidx.transpose(1, 0, 2).reshape(b_sz, _NW * _L)
    mx = vals.max(axis=1, keepdims=True)
    best = jnp.where(vals == mx, idxs, jnp.int32(2**31 - 1)).min(axis=1)
    return x_grid[best] - rec_centroid[None, :]


# speedup vs baseline: 1.5963x; 1.0770x over previous
import functools

import numpy as np
import jax
import jax.numpy as jnp
from jax import lax
from jax.experimental import pallas as pl
from jax.experimental.pallas import tpu as pltpu
from jax.experimental.pallas import tpu_sc as plsc

_SR_MAX_TAU = 40
_LAGS = 2 * _SR_MAX_TAU + 1
_T = 4096
_KF = _T // 2 + 1
_KPAD = 17 * 128
_LPAD = 128
_NW = 32
_L = 16
_QB = 8


def _idft_consts():
    k = np.arange(_KF)
    t = (np.arange(_LAGS) - _SR_MAX_TAU) % _T
    theta = 2.0 * np.pi * ((k[:, None] * t[None, :]) % _T) / _T
    w = np.full((_KF, 1), 2.0)
    w[0, 0] = 1.0
    w[-1, 0] = 1.0
    c = (w * np.cos(theta)) / _T
    s = (-w * np.sin(theta)) / _T
    cp = np.zeros((_KPAD, _LPAD), np.float32)
    sp = np.zeros((_KPAD, _LPAD), np.float32)
    cp[:_KF, :_LAGS] = c
    sp[:_KF, :_LAGS] = s
    return cp, sp


def _pairs(x, m):
    a = jnp.concatenate(
        [jnp.broadcast_to(x[k:k + 1], (m - 1 - k, x.shape[1]))
         for k in range(m - 1)], axis=0)
    b = jnp.concatenate([x[k + 1:m] for k in range(m - 1)], axis=0)
    return a, b


def _whiten_idft_body(xr_ref, xi_ref, c_ref, s_ref, out_ref):
    m = xr_ref.shape[1]
    prs, pis = [], []
    for q in range(_QB):
        xr = xr_ref[q]
        xi = xi_ref[q]
        ar, br = _pairs(xr, m)
        ai, bi = _pairs(xi, m)
        gr = ar * br + ai * bi
        gi = ai * br - ar * bi
        inv = 1.0 / (jnp.sqrt(gr * gr + gi * gi) + 1e-12)
        prs.append(gr * inv)
        pis.append(gi * inv)
    pr = jnp.concatenate(prs, axis=0)
    pi = jnp.concatenate(pis, axis=0)
    out_ref[:, :] = (
        jnp.dot(pr, c_ref[:, :], precision=lax.Precision.HIGHEST,
                preferred_element_type=jnp.float32)
        + jnp.dot(pi, s_ref[:, :], precision=lax.Precision.HIGHEST,
                  preferred_element_type=jnp.float32)
    )


def _whiten_idft(xr, xi, cmat, smat):
    b_sz, m_sz, _ = xr.shape
    p_sz = m_sz * (m_sz - 1) // 2
    rows_blk = _QB * p_sz
    spec_x = pl.BlockSpec((_QB, m_sz, _KPAD), lambda i: (i, 0, 0))
    spec_c = pl.BlockSpec((_KPAD, _LPAD), lambda i: (0, 0))
    return pl.pallas_call(
        _whiten_idft_body,
        grid=(b_sz // _QB,),
        in_specs=[spec_x, spec_x, spec_c, spec_c],
        out_specs=pl.BlockSpec((rows_blk, _LPAD), lambda i: (i, 0)),
        out_shape=jax.ShapeDtypeStruct((b_sz * p_sz, _LPAD), jnp.float32),
    )(xr, xi, cmat, smat)


def _make_sc_search(n_grid, n_pairs, batch, table_cols):
    npw_raw = -(-n_grid // _NW)
    npw = -(-npw_raw // _L) * _L
    chunks = npw // _L
    n_pad = npw * _NW
    mesh = plsc.VectorSubcoreMesh(core_axis_name="c", subcore_axis_name="s")

    @functools.partial(
        pl.kernel,
        mesh=mesh,
        compiler_params=pltpu.CompilerParams(needs_layout_passes=False),
        out_type=(
            jax.ShapeDtypeStruct((_NW, batch, _L), jnp.float32),
            jax.ShapeDtypeStruct((_NW, batch, _L), jnp.int32),
        ),
        scratch_types=[
            pltpu.VMEM((batch * table_cols,), jnp.float32),
            pltpu.VMEM((npw * n_pairs,), jnp.int32),
            pltpu.VMEM((batch, _L), jnp.float32),
            pltpu.VMEM((batch, _L), jnp.int32),
        ],
    )
    def sc_search(table_hbm, tau_hbm, omax_hbm, oidx_hbm,
                  table_v, tau_v, rmax_v, ridx_v):
        wid = lax.axis_index("s") * 2 + lax.axis_index("c")
        base = lax.min(wid * npw, n_grid - npw)
        pltpu.sync_copy(table_hbm, table_v)
        pltpu.sync_copy(tau_hbm.at[pl.ds(base * n_pairs, npw * n_pairs)],
                        tau_v)
        for b in range(batch):
            rmax_v[b, :] = jnp.full((_L,), -jnp.inf, jnp.float32)
            ridx_v[b, :] = jnp.zeros((_L,), jnp.int32)
        lane28 = lax.iota(jnp.int32, _L) * n_pairs

        def chunk_body(c, _):
            posbase = lane28 + c * (_L * n_pairs)

            def pair_body(ph, accs):
                p0 = 2 * ph
                tv0 = plsc.load_gather(tau_v, [posbase + p0])
                tv1 = plsc.load_gather(tau_v, [posbase + (p0 + 1)])
                idx0 = tv0 + p0 * _LAGS
                idx1 = tv1 + p0 * _LAGS + _LAGS
                accs = tuple(
                    accs[b] + plsc.load_gather(table_v, [idx0 + b * table_cols])
                    for b in range(batch)
                )
                return tuple(
                    accs[b] + plsc.load_gather(table_v, [idx1 + b * table_cols])
                    for b in range(batch)
                )

            zeros = tuple(jnp.zeros((_L,), jnp.float32) for _ in range(batch))
            accs = lax.fori_loop(0, n_pairs // 2, pair_body, zeros)
            nvec = base + c * _L + lax.iota(jnp.int32, _L)
            for b in range(batch):
                m = rmax_v[b, :]
                upd = accs[b] > m
                rmax_v[b, :] = jnp.where(upd, accs[b], m)
                ridx_v[b, :] = jnp.where(upd, nvec, ridx_v[b, :])
            return _

        lax.fori_loop(0, chunks, chunk_body, None)
        pltpu.sync_copy(rmax_v, omax_hbm.at[wid])
        pltpu.sync_copy(ridx_v, oidx_hbm.at[wid])

    return sc_search, npw, n_pad


def kernel(signal, x_grid, rec_centroid, tau, combinations):
    del combinations
    b_sz, m_sz, t_sz = signal.shape
    p_sz = m_sz * (m_sz - 1) // 2
    n_grid = tau.shape[0]
    table_cols = p_sz * _LAGS

    spec = jnp.fft.rfft(signal, axis=-1)
    xr = jnp.pad(jnp.real(spec).astype(jnp.float32),
                 ((0, 0), (0, 0), (0, _KPAD - _KF)))
    xi = jnp.pad(jnp.imag(spec).astype(jnp.float32),
                 ((0, 0), (0, 0), (0, _KPAD - _KF)))

    cnp, snp = _idft_consts()
    cc = _whiten_idft(xr, xi, jnp.asarray(cnp), jnp.asarray(snp))
    table = cc[:, :_LAGS].reshape(b_sz * table_cols)

    sc_search, npw, n_pad = _make_sc_search(n_grid, p_sz, b_sz, table_cols)
    pmax, pidx = sc_search(table, tau.reshape(-1))

    vals = pmax.transpose(1, 0, 2).reshape(b_sz, _NW * _L)
    idxs = pidx.transpose(1, 0, 2).reshape(b_sz, _NW * _L)
    mx = vals.max(axis=1, keepdims=True)
    best = jnp.where(vals == mx, idxs, jnp.int32(2**31 - 1)).min(axis=1)
    return x_grid[best] - rec_centroid[None, :]
